# baseline (speedup 1.0000x reference)
import functools

import jax
import jax.numpy as jnp
from jax import lax
from jax.experimental import pallas as pl

EMB = 256
NG = 64


def _cdiv(a, b):
    return (a + b - 1) // b


def _mm_body(a_ref, w_ref, b_ref, o_ref, *, relu):
    acc = jnp.dot(a_ref[...], w_ref[...], preferred_element_type=jnp.float32)
    acc = acc + b_ref[...]
    if relu:
        acc = jnp.maximum(acc, 0.0)
    o_ref[...] = acc


def _mm(a, w, b=None, relu=False, bm=512):
    m, k = a.shape
    n = w.shape[1]
    if b is None:
        b = jnp.zeros((n,), jnp.float32)
    b2d = b.reshape(1, n)
    grid = (_cdiv(m, bm),)
    return pl.pallas_call(
        functools.partial(_mm_body, relu=relu),
        grid=grid,
        in_specs=[
            pl.BlockSpec((bm, k), lambda i: (i, 0)),
            pl.BlockSpec((k, n), lambda i: (0, 0)),
            pl.BlockSpec((1, n), lambda i: (0, 0)),
        ],
        out_specs=pl.BlockSpec((bm, n), lambda i: (i, 0)),
        out_shape=jax.ShapeDtypeStruct((m, n), jnp.float32),
    )(a, w, b2d)


def _coef_body(ha_ref, hb_ref, qa_ref, qb_ref, q0_ref, p2_ref, pb2_ref, o_ref):
    t = (jnp.dot(ha_ref[...], qa_ref[...], preferred_element_type=jnp.float32)
         + jnp.dot(hb_ref[...], qb_ref[...], preferred_element_type=jnp.float32)
         + q0_ref[...])
    t = jnp.maximum(t, 0.0)
    o_ref[...] = jnp.dot(t, p2_ref[...], preferred_element_type=jnp.float32) + pb2_ref[...]


def _coef_mlp(ha, hb, qa, qb, q0, p2, pb2, bm=1024):
    m = ha.shape[0]
    grid = (_cdiv(m, bm),)
    return pl.pallas_call(
        _coef_body,
        grid=grid,
        in_specs=[
            pl.BlockSpec((bm, 128), lambda i: (i, 0)),
            pl.BlockSpec((bm, 128), lambda i: (i, 0)),
            pl.BlockSpec((128, EMB), lambda i: (0, 0)),
            pl.BlockSpec((128, EMB), lambda i: (0, 0)),
            pl.BlockSpec((1, EMB), lambda i: (0, 0)),
            pl.BlockSpec((EMB, 1), lambda i: (0, 0)),
            pl.BlockSpec((1, 1), lambda i: (0, 0)),
        ],
        out_specs=pl.BlockSpec((bm, 1), lambda i: (i, 0)),
        out_shape=jax.ShapeDtypeStruct((m, 1), jnp.float32),
    )(ha, hb, qa, qb, q0.reshape(1, EMB), p2, pb2.reshape(1, 1))


def _upd_body(h_ref, s1_ref, s2_ref, se_ref, ss_ref, d_ref, w_ref, ub1_ref,
              u2_ref, ub2_ref, o_ref):
    E = EMB
    z = (jnp.dot(h_ref[...], w_ref[0:E, :], preferred_element_type=jnp.float32)
         + jnp.dot(s1_ref[...], w_ref[E:2 * E, :], preferred_element_type=jnp.float32)
         + jnp.dot(s2_ref[...], w_ref[2 * E:3 * E, :], preferred_element_type=jnp.float32)
         + jnp.dot(se_ref[...], w_ref[3 * E:4 * E, :], preferred_element_type=jnp.float32)
         + jnp.dot(ss_ref[...], w_ref[4 * E:5 * E, :], preferred_element_type=jnp.float32)
         + jnp.dot(d_ref[...], w_ref[5 * E:5 * E + 8, :], preferred_element_type=jnp.float32)
         + ub1_ref[...])
    z = jnp.maximum(z, 0.0)
    o_ref[...] = (h_ref[...]
                  + jnp.dot(z, u2_ref[...], preferred_element_type=jnp.float32)
                  + ub2_ref[...])


def _upd_mlp(h0, s1, s2, se, ss, d8, wcomb, ub1, u2, ub2, bm=512):
    m = h0.shape[0]
    grid = (_cdiv(m, bm),)
    spec_e = pl.BlockSpec((bm, EMB), lambda i: (i, 0))
    return pl.pallas_call(
        _upd_body,
        grid=grid,
        in_specs=[
            spec_e, spec_e, spec_e, spec_e, spec_e,
            pl.BlockSpec((bm, 8), lambda i: (i, 0)),
            pl.BlockSpec((5 * EMB + 8, EMB), lambda i: (0, 0)),
            pl.BlockSpec((1, EMB), lambda i: (0, 0)),
            pl.BlockSpec((EMB, EMB), lambda i: (0, 0)),
            pl.BlockSpec((1, EMB), lambda i: (0, 0)),
        ],
        out_specs=spec_e,
        out_shape=jax.ShapeDtypeStruct((m, EMB), jnp.float32),
    )(h0, s1, s2, se, ss, d8, wcomb, ub1.reshape(1, EMB), u2, ub2.reshape(1, EMB))


def kernel(x, pos, edge_attr, sse_attr, batch_idx, adj1_index, adj2_index,
           n1_edge, n1_node, n2_sse, n2_node, params):
    N = x.shape[0]
    E = EMB
    d1, s1 = adj1_index[0], adj1_index[1]
    d2, s2 = adj2_index[0], adj2_index[1]

    H0 = _mm(x, params["emb0_w"], params["emb0_b"])
    X = pos

    deg1 = jax.ops.segment_sum(jnp.ones(d1.shape, jnp.float32), d1, num_segments=N)
    deg2 = jax.ops.segment_sum(jnp.ones(d2.shape, jnp.float32), d2, num_segments=N)
    dege = jax.ops.segment_sum(jnp.ones(n1_node.shape, jnp.float32), n1_node, num_segments=N)
    degs = jax.ops.segment_sum(jnp.ones(n2_node.shape, jnp.float32), n2_node, num_segments=N)
    d8 = jnp.stack([deg1, deg2, dege, degs] + [jnp.zeros((N,), jnp.float32)] * 4, axis=1)

    for lp in params["layers"]:
        W1_1, b1_1, W2_1, b2_1 = lp["msg1"]
        W1_2, b1_2, W2_2, b2_2 = lp["msg2"]
        W1_e, b1_e, W2_e, b2_e = lp["msge"]
        W1_s, b1_s, W2_s, b2_s = lp["msgs"]
        U1, ub1, U2, ub2 = lp["upd"]
        P1, pb1, P2, pb2 = lp["posm"]

        wbig = jnp.concatenate(
            [W1_1[:E], W1_1[E:2 * E], W1_2[:E], W1_2[E:2 * E], W1_e[:E], W1_s[:E]], axis=1)
        bbig = jnp.concatenate(
            [b1_1, jnp.zeros_like(b1_1), b1_2, jnp.zeros_like(b1_2),
             jnp.zeros_like(b1_e), jnp.zeros_like(b1_s)])
        P = _mm(H0, wbig, bbig)
        A1, B1 = P[:, 0:E], P[:, E:2 * E]
        A2, B2 = P[:, 2 * E:3 * E], P[:, 3 * E:4 * E]
        Ae, As = P[:, 4 * E:5 * E], P[:, 5 * E:6 * E]
        w1c_1, w1c_2 = W1_1[2 * E], W1_2[2 * E]
        Eproj = _mm(edge_attr, W1_e[E:], b1_e, bm=2048)
        Sproj = _mm(sse_attr, W1_s[E:], b1_s, bm=400)

        rel1 = X[d1] - X[s1]
        sq1 = jnp.sum(rel1 * rel1, axis=-1)
        rel2 = X[d2] - X[s2]
        sq2 = jnp.sum(rel2 * rel2, axis=-1)

        h1 = jnp.maximum(A1[d1] + B1[s1] + sq1[:, None] * w1c_1, 0.0)
        S1 = jax.ops.segment_sum(h1, d1, num_segments=N)
        h2 = jnp.maximum(A2[d2] + B2[s2] + sq2[:, None] * w1c_2, 0.0)
        S2 = jax.ops.segment_sum(h2, d2, num_segments=N)
        he = jnp.maximum(Ae[n1_node] + Eproj[n1_edge], 0.0)
        Se = jax.ops.segment_sum(he, n1_node, num_segments=N)
        hs = jnp.maximum(As[n2_node] + Sproj[n2_sse], 0.0)
        Ss = jax.ops.segment_sum(hs, n2_node, num_segments=N)

        wcomb = jnp.concatenate(
            [U1[:E], W2_1 @ U1[E:2 * E], W2_2 @ U1[2 * E:3 * E],
             W2_e @ U1[3 * E:4 * E], W2_s @ U1[4 * E:],
             jnp.stack([b2_1 @ U1[E:2 * E], b2_2 @ U1[2 * E:3 * E],
                        b2_e @ U1[3 * E:4 * E], b2_s @ U1[4 * E:]]
                       + [jnp.zeros((E,), jnp.float32)] * 4)], axis=0)
        H0new = _upd_mlp(H0, S1, S2, Se, Ss, d8, wcomb, ub1, U2, ub2)

        Q = W2_1 @ P1
        q0 = b2_1 @ P1 + pb1
        coef = _coef_mlp(h1[:, :128], h1[:, 128:], Q[:128], Q[128:], q0, P2, pb2)
        agg = jax.ops.segment_sum(rel1 * coef, d1, num_segments=N)
        X = X + agg / (deg1 + 1.0)[:, None]
        H0 = H0new

    counts = jax.ops.segment_sum(jnp.ones((N,), jnp.float32), batch_idx, num_segments=NG)
    graph_emb = jax.ops.segment_sum(H0, batch_idx, num_segments=NG) / jnp.maximum(counts, 1.0)[:, None]
    return (H0, graph_emb, X)
